# Initial kernel scaffold; baseline (speedup 1.0000x reference)
import jax
import jax.numpy as jnp
from jax.experimental import pallas as pl


def kernel(slots, indices, down_proj_values, up_proj_values):
    raise NotImplementedError("write your pallas kernel here")


# speedup vs baseline: 1.8891x; 1.8891x over previous
import math

import jax
import jax.numpy as jnp
from jax.experimental import pallas as pl

DIM = 2048
RANK = 32
NUM_ENTRIES = 64
_SCALE = 2.0 / math.sqrt(RANK)

_E_BLK = 8
_P = 256


def _adalora_block(idx_ref, s_ref, d_ref, u_ref, o_ref):
    j = pl.program_id(0)
    e0 = j * _E_BLK
    y = jnp.dot(s_ref[...], d_ref[...], preferred_element_type=jnp.float32)
    eid = jax.lax.broadcasted_iota(jnp.int32, (_P, _E_BLK * RANK), 1) // RANK + e0
    keep = eid == idx_ref[...]
    y = jnp.where(keep, y, 0.0) * _SCALE
    z = jnp.dot(y, u_ref[...], preferred_element_type=jnp.float32)

    @pl.when(j == 0)
    def _init():
        o_ref[...] = z

    @pl.when(j > 0)
    def _acc():
        o_ref[...] += z


def kernel(slots, indices, down_proj_values, up_proj_values):
    b, k, d = slots.shape
    p = b * k
    s2 = slots.reshape(p, d)
    idx = indices.reshape(p, 1).astype(jnp.int32)
    d2 = jnp.transpose(down_proj_values, (1, 0, 2)).reshape(d, NUM_ENTRIES * RANK)
    u2 = up_proj_values.reshape(NUM_ENTRIES * RANK, d)

    out = pl.pallas_call(
        _adalora_block,
        grid=(NUM_ENTRIES // _E_BLK,),
        in_specs=[
            pl.BlockSpec((p, 1), lambda j: (0, 0)),
            pl.BlockSpec((p, d), lambda j: (0, 0)),
            pl.BlockSpec((d, _E_BLK * RANK), lambda j: (0, j)),
            pl.BlockSpec((_E_BLK * RANK, d), lambda j: (j, 0)),
        ],
        out_specs=pl.BlockSpec((p, d), lambda j: (0, 0)),
        out_shape=jax.ShapeDtypeStruct((p, d), jnp.float32),
    )(idx, s2, d2, u2)
    return out.reshape(b, k, d)
